# Initial kernel scaffold; baseline (speedup 1.0000x reference)
import functools

import jax
import jax.numpy as jnp
from jax import lax
from jax.experimental import pallas as pl
from jax.experimental.pallas import tpu as pltpu
from jax.experimental.pallas import tpu_sc as plsc

NC = 2
NS = 16
NW = NC * NS

E = 320000
D = 128
N_NODES = 10000

EPW = E // NW
C = 80
NCHUNK = EPW // C


def _body(x_hbm, ei_hbm, out_hbm, idx_v, rows0, rows1, sem):
    wid = lax.axis_index("s") * NC + lax.axis_index("c")
    base = pl.multiple_of(wid * EPW, EPW)

    pltpu.sync_copy(ei_hbm.at[:, pl.ds(base, EPW)], idx_v)

    def chunk_step(j, _):
        off = pl.multiple_of(j * C, C)
        g0 = pltpu.async_copy(x_hbm.at[idx_v.at[0, pl.ds(off, C)]], rows0, sem)
        g1 = pltpu.async_copy(x_hbm.at[idx_v.at[1, pl.ds(off, C)]], rows1, sem)
        g0.wait()
        g1.wait()

        def add_row(r, _):
            for g in range(D // 16):
                sl = pl.ds(g * 16, 16)
                rows0[r, sl] = rows0[r, sl] + rows1[r, sl]
            return 0

        lax.fori_loop(0, C, add_row, 0)
        pltpu.sync_copy(rows0, out_hbm.at[pl.ds(base + off, C)])
        return 0

    lax.fori_loop(0, NCHUNK, chunk_step, 0)


def kernel(x, edge_index):
    ei = edge_index.astype(jnp.int32)
    mesh = plsc.VectorSubcoreMesh(core_axis_name="c", subcore_axis_name="s")
    run = pl.kernel(
        _body,
        mesh=mesh,
        out_type=jax.ShapeDtypeStruct((E, D), jnp.float32),
        scratch_types=[
            pltpu.VMEM((2, EPW), jnp.int32),
            pltpu.VMEM((C, D), jnp.float32),
            pltpu.VMEM((C, D), jnp.float32),
            pltpu.SemaphoreType.DMA,
        ],
    )
    return run(x, ei)


# speedup vs baseline: 6.7285x; 6.7285x over previous
import functools

import jax
import jax.numpy as jnp
from jax import lax
from jax.experimental import pallas as pl
from jax.experimental.pallas import tpu as pltpu
from jax.experimental.pallas import tpu_sc as plsc

NC = 2
NS = 16
NW = NC * NS

E = 320000
D = 128
N_NODES = 10000

EPW = E // NW
C = 80
NCHUNK = EPW // C


def _body(x_hbm, ei_hbm, out_hbm, idx0_v, idx1_v, rows0, rows1, sem):
    wid = lax.axis_index("s") * NC + lax.axis_index("c")
    base = pl.multiple_of(wid * EPW, EPW)

    pltpu.sync_copy(ei_hbm.at[pl.ds(base, EPW)], idx0_v)
    pltpu.sync_copy(ei_hbm.at[pl.ds(E + base, EPW)], idx1_v)

    def chunk_step(j, _):
        off = pl.multiple_of(j * C, C)
        g0 = pltpu.async_copy(x_hbm.at[idx0_v.at[pl.ds(off, C)]], rows0, sem)
        g1 = pltpu.async_copy(x_hbm.at[idx1_v.at[pl.ds(off, C)]], rows1, sem)
        g0.wait()
        g1.wait()

        def add_row(r, _):
            for g in range(D // 16):
                sl = pl.ds(g * 16, 16)
                rows0[r, sl] = rows0[r, sl] + rows1[r, sl]
            return 0

        lax.fori_loop(0, C, add_row, 0)
        pltpu.sync_copy(rows0, out_hbm.at[pl.ds(base + off, C)])
        return 0

    lax.fori_loop(0, NCHUNK, chunk_step, 0)


def kernel(x, edge_index):
    ei = edge_index.astype(jnp.int32).reshape(-1)
    mesh = plsc.VectorSubcoreMesh(core_axis_name="c", subcore_axis_name="s")
    run = pl.kernel(
        _body,
        mesh=mesh,
        out_type=jax.ShapeDtypeStruct((E, D), jnp.float32),
        scratch_types=[
            pltpu.VMEM((EPW,), jnp.int32),
            pltpu.VMEM((EPW,), jnp.int32),
            pltpu.VMEM((C, D), jnp.float32),
            pltpu.VMEM((C, D), jnp.float32),
            pltpu.SemaphoreType.DMA,
        ],
    )
    return run(x, ei)


# speedup vs baseline: 9.0547x; 1.3457x over previous
import functools

import jax
import jax.numpy as jnp
from jax import lax
from jax.experimental import pallas as pl
from jax.experimental.pallas import tpu as pltpu
from jax.experimental.pallas import tpu_sc as plsc

NC = 2
NS = 16
NW = NC * NS

E = 320000
D = 128
N_NODES = 10000

EPW = E // NW
C = 80
NCHUNK = EPW // C


def _body(x_hbm, ei_hbm, out_hbm, x_sh, idx0_v, idx1_v, rows0, rows1, sem):
    sid = lax.axis_index("s")
    wid = sid * NC + lax.axis_index("c")
    base = pl.multiple_of(wid * EPW, EPW)

    @pl.when(sid < 5)
    def _stage():
        r0 = pl.multiple_of(sid * 2000, 2000)
        pltpu.sync_copy(x_hbm.at[pl.ds(r0, 2000)], x_sh.at[pl.ds(r0, 2000)])

    pltpu.sync_copy(ei_hbm.at[pl.ds(base, EPW)], idx0_v)
    pltpu.sync_copy(ei_hbm.at[pl.ds(E + base, EPW)], idx1_v)
    plsc.subcore_barrier()

    def chunk_step(j, _):
        off = pl.multiple_of(j * C, C)
        g0 = pltpu.async_copy(x_sh.at[idx0_v.at[pl.ds(off, C)]], rows0, sem)
        g1 = pltpu.async_copy(x_sh.at[idx1_v.at[pl.ds(off, C)]], rows1, sem)
        g0.wait()
        g1.wait()

        def add_row(r, _):
            for g in range(D // 16):
                sl = pl.ds(g * 16, 16)
                rows0[r, sl] = rows0[r, sl] + rows1[r, sl]
            return 0

        lax.fori_loop(0, C, add_row, 0)
        pltpu.sync_copy(rows0, out_hbm.at[pl.ds(base + off, C)])
        return 0

    lax.fori_loop(0, NCHUNK, chunk_step, 0)


def kernel(x, edge_index):
    ei = edge_index.astype(jnp.int32).reshape(-1)
    mesh = plsc.VectorSubcoreMesh(core_axis_name="c", subcore_axis_name="s")
    run = pl.kernel(
        _body,
        mesh=mesh,
        out_type=jax.ShapeDtypeStruct((E, D), jnp.float32),
        scratch_types=[
            pltpu.VMEM_SHARED((N_NODES, D), jnp.float32),
            pltpu.VMEM((EPW,), jnp.int32),
            pltpu.VMEM((EPW,), jnp.int32),
            pltpu.VMEM((C, D), jnp.float32),
            pltpu.VMEM((C, D), jnp.float32),
            pltpu.SemaphoreType.DMA,
        ],
    )
    return run(x, ei)


# speedup vs baseline: 12.4583x; 1.3759x over previous
import functools

import jax
import jax.numpy as jnp
from jax import lax
from jax.experimental import pallas as pl
from jax.experimental.pallas import tpu as pltpu
from jax.experimental.pallas import tpu_sc as plsc

NC = 2
NS = 16
NW = NC * NS

E = 320000
D = 128
N_NODES = 10000

EPW = E // NW
C = 80
NCHUNK = EPW // C


def _body(x_hbm, ei_hbm, out_hbm, x_sh, idx0_v, idx1_v, rows0, rows1, sem):
    sid = lax.axis_index("s")
    wid = sid * NC + lax.axis_index("c")
    base = pl.multiple_of(wid * EPW, EPW)

    @pl.when(sid < 5)
    def _stage():
        r0 = pl.multiple_of(sid * 2000, 2000)
        pltpu.sync_copy(x_hbm.at[pl.ds(r0, 2000)], x_sh.at[pl.ds(r0, 2000)])

    pltpu.sync_copy(ei_hbm.at[pl.ds(base, EPW)], idx0_v)
    pltpu.sync_copy(ei_hbm.at[pl.ds(E + base, EPW)], idx1_v)
    plsc.subcore_barrier()

    def chunk_step(j, _):
        off = pl.multiple_of(j * C, C)
        g0 = pltpu.async_copy(x_sh.at[idx0_v.at[pl.ds(off, C)]], rows0, sem)
        g0.wait()
        g1 = pltpu.async_copy(
            x_sh.at[idx1_v.at[pl.ds(off, C)]], rows0, sem, add=True)
        g1.wait()
        pltpu.sync_copy(rows0, out_hbm.at[pl.ds(base + off, C)])
        return 0

    lax.fori_loop(0, NCHUNK, chunk_step, 0)


def kernel(x, edge_index):
    ei = edge_index.astype(jnp.int32).reshape(-1)
    mesh = plsc.VectorSubcoreMesh(core_axis_name="c", subcore_axis_name="s")
    run = pl.kernel(
        _body,
        mesh=mesh,
        out_type=jax.ShapeDtypeStruct((E, D), jnp.float32),
        scratch_types=[
            pltpu.VMEM_SHARED((N_NODES, D), jnp.float32),
            pltpu.VMEM((EPW,), jnp.int32),
            pltpu.VMEM((EPW,), jnp.int32),
            pltpu.VMEM((C, D), jnp.float32),
            pltpu.VMEM((C, D), jnp.float32),
            pltpu.SemaphoreType.DMA,
        ],
    )
    return run(x, ei)


# speedup vs baseline: 17.4923x; 1.4041x over previous
import functools

import jax
import jax.numpy as jnp
from jax import lax
from jax.experimental import pallas as pl
from jax.experimental.pallas import tpu as pltpu
from jax.experimental.pallas import tpu_sc as plsc

NC = 2
NS = 16
NW = NC * NS

E = 320000
D = 128
N_NODES = 10000

EPW = E // NW
C = 40
NCHUNK = EPW // C


NSLOT = 4


def _body(x_hbm, ei_hbm, out_hbm, x_sh, idx0_v, idx1_v, rows,
          sem0, sem1, sem2, sem3):
    sems = [sem0, sem1, sem2, sem3]
    sid = lax.axis_index("s")
    wid = sid * NC + lax.axis_index("c")
    base = pl.multiple_of(wid * EPW, EPW)

    @pl.when(sid < 5)
    def _stage():
        r0 = pl.multiple_of(sid * 2000, 2000)
        pltpu.sync_copy(x_hbm.at[pl.ds(r0, 2000)], x_sh.at[pl.ds(r0, 2000)])

    pltpu.sync_copy(ei_hbm.at[pl.ds(base, EPW)], idx0_v)
    pltpu.sync_copy(ei_hbm.at[pl.ds(E + base, EPW)], idx1_v)
    plsc.subcore_barrier()

    def _wait(slot, dst_is_hbm):
        if dst_is_hbm:
            pltpu.make_async_copy(
                rows.at[slot], out_hbm.at[pl.ds(0, C)], sems[slot]).wait()
        else:
            pltpu.make_async_copy(
                x_hbm.at[pl.ds(0, C)], rows.at[slot], sems[slot]).wait()

    def group(t, _):
        for u in range(NSLOT):
            j = t * NSLOT + u

            @pl.when(jnp.logical_and(j >= NSLOT, j - NSLOT < NCHUNK))
            def _a():
                _wait(u, dst_is_hbm=True)

            @pl.when(j < NCHUNK)
            def _b():
                off = pl.multiple_of(j * C, C)
                pltpu.async_copy(
                    x_sh.at[idx0_v.at[pl.ds(off, C)]], rows.at[u], sems[u])

            u1 = (u - 1) % NSLOT

            @pl.when(jnp.logical_and(j >= 1, j <= NCHUNK))
            def _c():
                off = pl.multiple_of((j - 1) * C, C)
                _wait(u1, dst_is_hbm=False)
                pltpu.async_copy(
                    x_sh.at[idx1_v.at[pl.ds(off, C)]], rows.at[u1],
                    sems[u1], add=True)

            u2 = (u - 2) % NSLOT

            @pl.when(jnp.logical_and(j >= 2, j <= NCHUNK + 1))
            def _d():
                off = pl.multiple_of((j - 2) * C, C)
                _wait(u2, dst_is_hbm=False)
                pltpu.async_copy(
                    rows.at[u2], out_hbm.at[pl.ds(base + off, C)], sems[u2])
        return 0

    n_groups = (NCHUNK + NSLOT + NSLOT - 1) // NSLOT + 1
    lax.fori_loop(0, n_groups, group, 0)


def kernel(x, edge_index):
    ei = edge_index.astype(jnp.int32).reshape(-1)
    mesh = plsc.VectorSubcoreMesh(core_axis_name="c", subcore_axis_name="s")
    run = pl.kernel(
        _body,
        mesh=mesh,
        out_type=jax.ShapeDtypeStruct((E, D), jnp.float32),
        scratch_types=[
            pltpu.VMEM_SHARED((N_NODES, D), jnp.float32),
            pltpu.VMEM((EPW,), jnp.int32),
            pltpu.VMEM((EPW,), jnp.int32),
            pltpu.VMEM((NSLOT, C, D), jnp.float32),
            pltpu.SemaphoreType.DMA,
            pltpu.SemaphoreType.DMA,
            pltpu.SemaphoreType.DMA,
            pltpu.SemaphoreType.DMA,
        ],
    )
    return run(x, ei)
